# baseline (speedup 1.0000x reference)
import functools

import jax
import jax.numpy as jnp
from jax import lax
from jax.experimental import pallas as pl
from jax.experimental.pallas import tpu as pltpu
from jax.experimental.pallas import tpu_sc as plsc

_R = 32768
_C = 64
_NC = 2
_NS = 16
_L = 16
_NW = _NC * _NS
_RPW = _R // _NW
_NBLK = _RPW // _L

_mesh = plsc.VectorSubcoreMesh(core_axis_name="c", subcore_axis_name="s")


@functools.partial(
    pl.kernel,
    out_type=(
        jax.ShapeDtypeStruct((_R,), jnp.int32),
        jax.ShapeDtypeStruct((_R,), jnp.int32),
    ),
    mesh=_mesh,
    compiler_params=pltpu.CompilerParams(needs_layout_passes=False),
    scratch_types=[
        pltpu.VMEM((_RPW * _C,), jnp.float32),
        pltpu.VMEM((_RPW,), jnp.int32),
        pltpu.VMEM((_RPW,), jnp.int32),
    ],
)
def _route_mask_sc(route_hbm, nr_hbm, r_hbm, buf, nr_buf, r_buf):
    wid = lax.axis_index("s") * _NC + lax.axis_index("c")
    base = wid * (_RPW * _C)
    pltpu.sync_copy(route_hbm.at[pl.ds(base, _RPW * _C)], buf)

    row_off = lax.iota(jnp.int32, _L) * _C

    def body(b, carry):
        idx0 = row_off + b * (_L * _C)
        c0 = plsc.load_gather(buf, [idx0])
        acc = []
        for j in range(1, _C):
            g = plsc.load_gather(buf, [idx0 + j])
            if len(acc) < 8:
                acc.append(g)
            else:
                k = (j - 1) % 8
                acc[k] = jnp.maximum(acc[k], g)
        while len(acc) > 1:
            nxt = [jnp.maximum(acc[i], acc[i + 1])
                   for i in range(0, len(acc) - 1, 2)]
            if len(acc) % 2:
                nxt.append(acc[-1])
            acc = nxt
        second = jnp.where(acc[0] > c0, 1, 0).astype(jnp.int32)
        r_buf[pl.ds(b * _L, _L)] = second
        nr_buf[pl.ds(b * _L, _L)] = 1 - second
        return carry

    lax.fori_loop(0, _NBLK, body, 0)

    out_base = wid * _RPW
    pltpu.sync_copy(nr_buf, nr_hbm.at[pl.ds(out_base, _RPW)])
    pltpu.sync_copy(r_buf, r_hbm.at[pl.ds(out_base, _RPW)])


def kernel(route, skip_dim):
    nr, r = _route_mask_sc(route.reshape(-1))
    cond = skip_dim == 1
    first = jnp.where(cond, nr, r).astype(jnp.bool_)
    second = jnp.where(cond, r, nr).astype(jnp.bool_)
    return (first, second)


# speedup vs baseline: 1.5150x; 1.5150x over previous
import functools

import jax
import jax.numpy as jnp
from jax import lax
from jax.experimental import pallas as pl
from jax.experimental.pallas import tpu as pltpu
from jax.experimental.pallas import tpu_sc as plsc

_R = 32768
_C = 64
_NC = 2
_NS = 16
_L = 16
_NW = _NC * _NS
_RPW = _R // _NW
_NBLK = _RPW // _L

_mesh = plsc.VectorSubcoreMesh(core_axis_name="c", subcore_axis_name="s")


@functools.partial(
    pl.kernel,
    out_type=(
        jax.ShapeDtypeStruct((_R,), jnp.int32),
        jax.ShapeDtypeStruct((_R,), jnp.int32),
    ),
    mesh=_mesh,
    compiler_params=pltpu.CompilerParams(needs_layout_passes=False),
    scratch_types=[
        pltpu.VMEM((_RPW * _C,), jnp.float32),
        pltpu.VMEM((_RPW,), jnp.int32),
        pltpu.VMEM((_RPW,), jnp.int32),
    ],
)
def _route_mask_sc(route_hbm, nr_hbm, r_hbm, buf, nr_buf, r_buf):
    wid = lax.axis_index("s") * _NC + lax.axis_index("c")
    base = wid * (_RPW * _C)
    pltpu.sync_copy(route_hbm.at[pl.ds(base, _RPW * _C)], buf)

    iota = lax.iota(jnp.int32, _L)
    diag0 = iota * _C + iota
    wrapv = {j: jnp.where(iota >= _C - j, j - _C, j).astype(jnp.int32)
             for j in range(_C - _L + 1, _C)}

    def body(b, carry):
        dbase = diag0 + b * (_L * _C)
        g0 = plsc.load_gather(buf, [dbase])
        acc = [g0, None, None, None]
        c0 = g0
        for j in range(1, _C):
            off = wrapv[j] if j in wrapv else j
            g = plsc.load_gather(buf, [dbase + off])
            k = j % 4
            acc[k] = g if acc[k] is None else jnp.maximum(acc[k], g)
            if j >= _C - _L + 1:
                c0 = jnp.where(iota == _C - j, g, c0)
        m = jnp.maximum(jnp.maximum(acc[0], acc[1]),
                        jnp.maximum(acc[2], acc[3]))
        second = jnp.where(m > c0, 1, 0).astype(jnp.int32)
        r_buf[pl.ds(b * _L, _L)] = second
        nr_buf[pl.ds(b * _L, _L)] = 1 - second
        return carry

    lax.fori_loop(0, _NBLK, body, 0)

    out_base = wid * _RPW
    pltpu.sync_copy(nr_buf, nr_hbm.at[pl.ds(out_base, _RPW)])
    pltpu.sync_copy(r_buf, r_hbm.at[pl.ds(out_base, _RPW)])


def kernel(route, skip_dim):
    nr, r = _route_mask_sc(route.reshape(-1))
    cond = skip_dim == 1
    first = jnp.where(cond, nr, r).astype(jnp.bool_)
    second = jnp.where(cond, r, nr).astype(jnp.bool_)
    return (first, second)


# speedup vs baseline: 1.7301x; 1.1420x over previous
import functools

import jax
import jax.numpy as jnp
from jax import lax
from jax.experimental import pallas as pl
from jax.experimental.pallas import tpu as pltpu
from jax.experimental.pallas import tpu_sc as plsc

_R = 32768
_C = 64
_NC = 2
_NS = 16
_L = 16
_NW = _NC * _NS
_RPW = _R // _NW
_NBLK = _RPW // _L

_mesh = plsc.VectorSubcoreMesh(core_axis_name="c", subcore_axis_name="s")


@functools.partial(
    pl.kernel,
    out_type=(
        jax.ShapeDtypeStruct((_R,), jnp.int32),
        jax.ShapeDtypeStruct((_R,), jnp.int32),
    ),
    mesh=_mesh,
    compiler_params=pltpu.CompilerParams(needs_layout_passes=False),
    scratch_types=[
        pltpu.VMEM((_RPW * _C,), jnp.float32),
        pltpu.VMEM((_RPW,), jnp.int32),
        pltpu.VMEM((_RPW,), jnp.int32),
    ],
)
def _route_mask_sc(route_hbm, nr_hbm, r_hbm, buf, nr_buf, r_buf):
    wid = lax.axis_index("s") * _NC + lax.axis_index("c")
    base = wid * (_RPW * _C)
    pltpu.sync_copy(route_hbm.at[pl.ds(base, _RPW * _C)], buf)

    iota = lax.iota(jnp.int32, _L)
    lane0 = iota == 0

    @plsc.parallel_loop(0, _RPW, unroll=4)
    def _row(i):
        rbase = i * _C
        v0 = buf[pl.ds(rbase, _L)]
        v1 = buf[pl.ds(rbase + _L, _L)]
        v2 = buf[pl.ds(rbase + 2 * _L, _L)]
        v3 = buf[pl.ds(rbase + 3 * _L, _L)]
        m = jnp.max(jnp.maximum(jnp.maximum(v0, v1), jnp.maximum(v2, v3)))
        second = jnp.where(m > v0[0], 1, 0).astype(jnp.int32)
        sv = jnp.broadcast_to(second, (_L,))
        iv = jnp.broadcast_to(i, (_L,)).astype(jnp.int32)
        plsc.store_scatter(r_buf, [iv], sv, mask=lane0)
        plsc.store_scatter(nr_buf, [iv], 1 - sv, mask=lane0)

    out_base = wid * _RPW
    pltpu.sync_copy(nr_buf, nr_hbm.at[pl.ds(out_base, _RPW)])
    pltpu.sync_copy(r_buf, r_hbm.at[pl.ds(out_base, _RPW)])


def kernel(route, skip_dim):
    nr, r = _route_mask_sc(route.reshape(-1))
    cond = skip_dim == 1
    first = jnp.where(cond, nr, r).astype(jnp.bool_)
    second = jnp.where(cond, r, nr).astype(jnp.bool_)
    return (first, second)


# speedup vs baseline: 2.9463x; 1.7030x over previous
import functools

import jax
import jax.numpy as jnp
from jax import lax
from jax.experimental import pallas as pl
from jax.experimental.pallas import tpu as pltpu
from jax.experimental.pallas import tpu_sc as plsc

_T = 32768
_E = 64
_NC = 2
_NS = 16
_L = 16
_NW = _NC * _NS
_TPW = _T // _NW
_NG = _TPW // _L

_mesh = plsc.VectorSubcoreMesh(core_axis_name="c", subcore_axis_name="s")


@functools.partial(
    pl.kernel,
    out_type=(
        jax.ShapeDtypeStruct((_T,), jnp.int32),
        jax.ShapeDtypeStruct((_T,), jnp.int32),
    ),
    mesh=_mesh,
    compiler_params=pltpu.CompilerParams(
        needs_layout_passes=False,
        use_tc_tiling_on_sc=True,
    ),
    scratch_types=[
        pltpu.VMEM((_E, _TPW), jnp.float32),
        pltpu.VMEM((_TPW,), jnp.int32),
        pltpu.VMEM((_TPW,), jnp.int32),
    ],
)
def _route_mask_sc(routet_hbm, nr_hbm, r_hbm, buf, nr_buf, r_buf):
    wid = lax.axis_index("s") * _NC + lax.axis_index("c")
    tbase = wid * _TPW
    pltpu.sync_copy(routet_hbm.at[:, pl.ds(tbase, _TPW)], buf)

    @plsc.parallel_loop(0, _NG, unroll=2)
    def _grp(g):
        col = g * _L
        c0 = buf[0, pl.ds(col, _L)]
        acc = [c0, None, None, None]
        for e in range(1, _E):
            v = buf[e, pl.ds(col, _L)]
            k = e % 4
            acc[k] = v if acc[k] is None else jnp.maximum(acc[k], v)
        m = jnp.maximum(jnp.maximum(acc[0], acc[1]),
                        jnp.maximum(acc[2], acc[3]))
        second = jnp.where(m > c0, 1, 0).astype(jnp.int32)
        r_buf[pl.ds(col, _L)] = second
        nr_buf[pl.ds(col, _L)] = 1 - second

    pltpu.sync_copy(nr_buf, nr_hbm.at[pl.ds(tbase, _TPW)])
    pltpu.sync_copy(r_buf, r_hbm.at[pl.ds(tbase, _TPW)])


def kernel(route, skip_dim):
    nr, r = _route_mask_sc(route.T)
    cond = skip_dim == 1
    first = jnp.where(cond, nr, r).astype(jnp.bool_)
    second = jnp.where(cond, r, nr).astype(jnp.bool_)
    return (first, second)
